# Initial kernel scaffold; baseline (speedup 1.0000x reference)
import jax
import jax.numpy as jnp
from jax.experimental import pallas as pl


def kernel(x, weight):
    raise NotImplementedError("write your pallas kernel here")


# speedup vs baseline: 1.0063x; 1.0063x over previous
import jax
import jax.numpy as jnp
from jax.experimental import pallas as pl

_SEQ_BLOCK = 512


def _add_kernel(x_ref, w_ref, o_ref):
    o_ref[...] = x_ref[...] + w_ref[...][None, :, :]


def kernel(x, weight):
    batch, seq, dim = x.shape
    w = weight[:seq, :]
    blk = _SEQ_BLOCK if seq % _SEQ_BLOCK == 0 else seq
    grid = (seq // blk,)
    return pl.pallas_call(
        _add_kernel,
        grid=grid,
        in_specs=[
            pl.BlockSpec((batch, blk, dim), lambda i: (0, i, 0)),
            pl.BlockSpec((blk, dim), lambda i: (i, 0)),
        ],
        out_specs=pl.BlockSpec((batch, blk, dim), lambda i: (0, i, 0)),
        out_shape=jax.ShapeDtypeStruct((batch, seq, dim), x.dtype),
    )(x, w)
